# baseline (speedup 1.0000x reference)
import functools

import jax
import jax.numpy as jnp
from jax.experimental import pallas as pl
from jax.experimental.pallas import tpu as pltpu
from jax.experimental.pallas import tpu_sc as plsc

B = 16384
D = 64
LANES = 128
NC = 2
NS = 16
NW = NC * NS
RPW = B // NW
CHUNK = 128
NCHUNK = RPW // CHUNK
HALF = RPW // 2

NUM_USERS = 1000000
NUM_MOVIES = 100000
UB_ROWS = (NUM_USERS + LANES - 1) // LANES
MB_ROWS = (NUM_MOVIES + LANES - 1) // LANES


def _sc_gather_body(idx2_u, idx2_m, idxb_u, idxb_m,
                    lmf_uw, lmf_mw, mlp_uw, mlp_mw, lmf_ub, lmf_mb,
                    gu_out, gm_out, gmu_out, gmm_out, gub_out, gmb_out,
                    iu, im, ibu, ibm, buf_a, buf_b, sem_a, sem_b):
    wid = jax.lax.axis_index("s") * NC + jax.lax.axis_index("c")
    base = wid * RPW

    pltpu.sync_copy(idx2_u.at[pl.ds(wid * NCHUNK, NCHUNK)], iu)
    pltpu.sync_copy(idx2_m.at[pl.ds(wid * NCHUNK, NCHUNK)], im)
    pltpu.sync_copy(idxb_u.at[pl.ds(wid * NCHUNK, NCHUNK)], ibu)
    pltpu.sync_copy(idxb_m.at[pl.ds(wid * NCHUNK, NCHUNK)], ibm)

    stages = []
    for table, idx, out in ((lmf_uw, iu, gu_out), (mlp_uw, iu, gmu_out),
                            (lmf_mw, im, gm_out), (mlp_mw, im, gmm_out),
                            (lmf_ub, ibu, gub_out), (lmf_mb, ibm, gmb_out)):
        for h in range(2):
            stages.append((table, idx, out, h))

    def fire(stage, buf, sem):
        table, idx, _, h = stage
        return [
            pltpu.async_copy(table.at[idx.at[h * 2 + j]],
                             buf.at[pl.ds(j * CHUNK, CHUNK)], sem)
            for j in range(2)
        ]

    def drain_write(stage, buf, cps):
        _, _, out, h = stage
        for c in cps:
            c.wait()
        pltpu.sync_copy(buf, out.at[pl.ds(base + h * HALF, HALF)])

    bufs = (buf_a, buf_b)
    sems = (sem_a, sem_b)
    cps = [None, None]
    cps[0] = fire(stages[0], buf_a, sem_a)
    cps[1] = fire(stages[1], buf_b, sem_b)
    for s in range(len(stages)):
        slot = s % 2
        drain_write(stages[s], bufs[slot], cps[slot])
        if s + 2 < len(stages):
            cps[slot] = fire(stages[s + 2], bufs[slot], sems[slot])


_sc_gather = functools.partial(
    pl.kernel,
    out_type=[jax.ShapeDtypeStruct((B, LANES), jnp.float32)] * 6,
    mesh=plsc.VectorSubcoreMesh(
        core_axis_name="c", subcore_axis_name="s", num_cores=NC,
        num_subcores=NS),
    scratch_types=[
        pltpu.VMEM((NCHUNK, CHUNK), jnp.int32),
        pltpu.VMEM((NCHUNK, CHUNK), jnp.int32),
        pltpu.VMEM((NCHUNK, CHUNK), jnp.int32),
        pltpu.VMEM((NCHUNK, CHUNK), jnp.int32),
        pltpu.VMEM((HALF, LANES), jnp.float32),
        pltpu.VMEM((HALF, LANES), jnp.float32),
        pltpu.SemaphoreType.DMA,
        pltpu.SemaphoreType.DMA,
    ],
)(_sc_gather_body)


RB = 2048


def _tc_dense_body(users, movies, gu, gm, gmu, gmm, gub, gmb,
                   W1, b1, W2, b2, W3, b3, Wf, bf, out):
    u = users[...]
    m = movies[...]
    pu = (u & 1) == 1
    pm = (m & 1) == 1

    def half(g, p):
        return jnp.where(p, g[:, D:], g[:, :D])

    uw = half(gu[...], pu)
    mw = half(gm[...], pm)
    mlp_u = half(gmu[...], pu)
    mlp_m = half(gmm[...], pm)

    lane = jax.lax.broadcasted_iota(jnp.int32, (RB, LANES), 1)
    ub = jnp.sum(jnp.where(lane == (u & 127), gub[...], 0.0),
                 axis=1, keepdims=True)
    mb = jnp.sum(jnp.where(lane == (m & 127), gmb[...], 0.0),
                 axis=1, keepdims=True)

    lmf = jax.nn.sigmoid(jnp.sum(uw * mw, axis=1, keepdims=True) + ub + mb)

    h = jnp.dot(mlp_u, W1[0:D, :], preferred_element_type=jnp.float32)
    h += jnp.dot(mlp_m, W1[D:2 * D, :], preferred_element_type=jnp.float32)
    h = jax.nn.relu(h + b1[...])
    h = jax.nn.relu(jnp.dot(h, W2[...], preferred_element_type=jnp.float32)
                    + b2[...])
    mlp = jax.nn.sigmoid(
        jnp.dot(h, W3[...], preferred_element_type=jnp.float32) + b3[...])
    x = jax.nn.sigmoid(lmf * Wf[0, 0] + mlp * Wf[1, 0] + bf[0, 0])
    out[...] = x * 4.5 + 0.5


def _tc_dense(users, movies, gu, gm, gmu, gmm, gub, gmb,
              W1, b1, W2, b2, W3, b3, Wf, bf):
    row = lambda i: (i, 0)
    rep = lambda i: (0, 0)
    return pl.pallas_call(
        _tc_dense_body,
        grid=(B // RB,),
        in_specs=[
            pl.BlockSpec((RB, 1), row),
            pl.BlockSpec((RB, 1), row),
            pl.BlockSpec((RB, LANES), row),
            pl.BlockSpec((RB, LANES), row),
            pl.BlockSpec((RB, LANES), row),
            pl.BlockSpec((RB, LANES), row),
            pl.BlockSpec((RB, LANES), row),
            pl.BlockSpec((RB, LANES), row),
            pl.BlockSpec((2 * D, D), rep),
            pl.BlockSpec((1, D), rep),
            pl.BlockSpec((D, 16), rep),
            pl.BlockSpec((1, 16), rep),
            pl.BlockSpec((16, 1), rep),
            pl.BlockSpec((1, 1), rep),
            pl.BlockSpec((2, 1), rep),
            pl.BlockSpec((1, 1), rep),
        ],
        out_specs=pl.BlockSpec((RB, 1), row),
        out_shape=jax.ShapeDtypeStruct((B, 1), jnp.float32),
    )(users, movies, gu, gm, gmu, gmm, gub, gmb,
      W1, b1, W2, b2, W3, b3, Wf, bf)


def kernel(users, movies, lmf_user_w, lmf_user_b, lmf_movie_w, lmf_movie_b,
           mlp_user_w, mlp_movie_w, W1, b1, W2, b2, W3, b3, Wf, bf):
    users = users.astype(jnp.int32)
    movies = movies.astype(jnp.int32)
    grid2 = (B // CHUNK, CHUNK)
    idx2_u = (users >> 1).reshape(grid2)
    idx2_m = (movies >> 1).reshape(grid2)
    idxb_u = (users >> 7).reshape(grid2)
    idxb_m = (movies >> 7).reshape(grid2)

    ub_pad = jnp.pad(lmf_user_b.reshape(-1),
                     (0, UB_ROWS * LANES - NUM_USERS)).reshape(UB_ROWS, LANES)
    mb_pad = jnp.pad(lmf_movie_b.reshape(-1),
                     (0, MB_ROWS * LANES - NUM_MOVIES)).reshape(MB_ROWS, LANES)

    gu, gm, gmu, gmm, gub, gmb = _sc_gather(
        idx2_u, idx2_m, idxb_u, idxb_m,
        lmf_user_w.reshape(NUM_USERS // 2, 2 * D),
        lmf_movie_w.reshape(NUM_MOVIES // 2, 2 * D),
        mlp_user_w.reshape(NUM_USERS // 2, 2 * D),
        mlp_movie_w.reshape(NUM_MOVIES // 2, 2 * D),
        ub_pad, mb_pad)
    return _tc_dense(
        users.reshape(B, 1), movies.reshape(B, 1),
        gu, gm, gmu, gmm, gub, gmb,
        W1, b1.reshape(1, D), W2, b2.reshape(1, 16), W3, b3.reshape(1, 1),
        Wf, bf.reshape(1, 1))
